# baseline (speedup 1.0000x reference)
import functools

import jax
import jax.numpy as jnp
from jax import lax
from jax.experimental import pallas as pl
from jax.experimental.pallas import tpu as pltpu
from jax.experimental.pallas import tpu_sc as plsc

T = 2048
H = 2048
FFN = 1408
E = 16
NG = 4
GSZ = E // NG
TOPK = 2
SCALE = 1.0

BLK = 128
MAXBLK = (T * TOPK) // BLK + E
MAXROWS = MAXBLK * BLK

NEG = -1e30


def _router_body(x_ref, gw_ref, dst_ref, wp_ref, be_ref, nb_ref):
    x = x_ref[...]
    gw = gw_ref[...]
    logits = lax.dot_general(x, gw, (((1,), (1,)), ((), ())),
                             preferred_element_type=jnp.float32)
    col = lax.broadcasted_iota(jnp.int32, (T, E), 1)
    gid = col // GSZ
    gmax_b = jnp.zeros((T, E), jnp.float32)
    for g in range(NG):
        m = jnp.max(jnp.where(gid == g, logits, NEG), axis=1, keepdims=True)
        gmax_b = jnp.where(gid == g, m, gmax_b)
    v1 = jnp.max(gmax_b, axis=1, keepdims=True)
    c1 = jnp.min(jnp.where(gmax_b == v1, col, E), axis=1, keepdims=True)
    g1 = c1 // GSZ
    masked2 = jnp.where(gid == g1, NEG, gmax_b)
    v2 = jnp.max(masked2, axis=1, keepdims=True)
    c2 = jnp.min(jnp.where(masked2 == v2, col, E), axis=1, keepdims=True)
    g2 = c2 // GSZ
    group_mask = (gid == g1) | (gid == g2)
    s = jnp.where(group_mask, logits, 0.0)
    m = jnp.max(s, axis=1, keepdims=True)
    ex = jnp.exp(s - m)
    p = ex / jnp.sum(ex, axis=1, keepdims=True)
    w1 = jnp.max(p, axis=1, keepdims=True)
    i1 = jnp.min(jnp.where(p == w1, col, E), axis=1, keepdims=True)
    p2 = jnp.where(col == i1, -1.0, p)
    w2 = jnp.max(p2, axis=1, keepdims=True)
    i2 = jnp.min(jnp.where(p2 == w2, col, E), axis=1, keepdims=True)

    oh1 = (i1 == col).astype(jnp.float32)
    oh2 = (i2 == col).astype(jnp.float32)
    oh = jnp.concatenate([oh1, oh2], axis=0)
    C = 128
    NC = (2 * T) // C
    r = lax.broadcasted_iota(jnp.int32, (C, C), 0)
    c = lax.broadcasted_iota(jnp.int32, (C, C), 1)
    tri = (c < r).astype(jnp.float32)
    locals_, totals_ = [], []
    for k in range(NC):
        chunk = oh[k * C:(k + 1) * C]
        locals_.append(jnp.dot(tri, chunk, preferred_element_type=jnp.float32))
        totals_.append(jnp.sum(chunk, axis=0, keepdims=True))
    totals = jnp.concatenate(totals_, axis=0)
    rr = lax.broadcasted_iota(jnp.int32, (NC, NC), 0)
    cc = lax.broadcasted_iota(jnp.int32, (NC, NC), 1)
    tri2 = (cc < rr).astype(jnp.float32)
    pref = jnp.dot(tri2, totals, preferred_element_type=jnp.float32)
    cum_ex = jnp.concatenate(
        [locals_[k] + pref[k:k + 1, :] for k in range(NC)], axis=0)
    rank = jnp.sum(cum_ex * oh, axis=1, keepdims=True)
    counts = jnp.sum(oh, axis=0, keepdims=True)
    padded = jnp.ceil(counts / BLK) * BLK
    jr = lax.broadcasted_iota(jnp.int32, (E, E), 0)
    ec = lax.broadcasted_iota(jnp.int32, (E, E), 1)
    off_r = jnp.sum(jnp.where(jr < ec, padded[0][:, None], 0.0),
                    axis=0, keepdims=True)
    ends_r = off_r + padded
    total_rows = jnp.sum(padded)
    dst = jnp.sum(off_r * oh, axis=1, keepdims=True) + rank
    dst_ref[...] = dst.astype(jnp.int32)
    wp_ref[...] = jnp.concatenate([w1, w2], axis=0) * SCALE
    nb = (total_rows / BLK).astype(jnp.int32)
    bi = lax.broadcasted_iota(jnp.int32, (MAXBLK, E), 0)
    be_raw = jnp.sum((ends_r <= bi.astype(jnp.float32) * BLK).astype(jnp.int32),
                     axis=1, keepdims=True)
    brow = lax.broadcasted_iota(jnp.int32, (MAXBLK, 1), 0)
    last_e = jnp.sum(jnp.where(brow == nb - 1, be_raw, 0))
    be_ref[...] = jnp.where(brow < nb, be_raw, last_e)
    nb_ref[...] = jnp.full((1, 1), nb, jnp.int32)


def _router(x, gate_w):
    return pl.pallas_call(
        _router_body,
        out_shape=[
            jax.ShapeDtypeStruct((2 * T, 1), jnp.int32),
            jax.ShapeDtypeStruct((2 * T, 1), jnp.float32),
            jax.ShapeDtypeStruct((MAXBLK, 1), jnp.int32),
            jax.ShapeDtypeStruct((1, 1), jnp.int32),
        ],
        compiler_params=pltpu.CompilerParams(
            vmem_limit_bytes=100 * 1024 * 1024),
    )(x, gate_w)


_CH = 16


_NCH = (2 * T) // 32 // _CH


def _dispatch_body(x_hbm, dst3_hbm, dst_hbm, wp_hbm, xs_hbm, wrow_hbm,
                   dstb_v, idx0_v, idx1_v, rows0_v, rows1_v,
                   dall_v, wall_v, wbuf_v, sem_g, sem_s):
    cid = lax.axis_index("c")
    sid = lax.axis_index("s")
    wid = sid * 2 + cid
    base = wid * _NCH * _CH
    pltpu.sync_copy(dst3_hbm.at[pl.ds(wid * _NCH, _NCH)], dstb_v)
    rows = (rows0_v, rows1_v)
    idxs = (idx0_v, idx1_v)
    gh, sh = {}, {}

    def start_gather(k):
        p = k & 1
        cb = base + k * _CH
        idxs[p][...] = (lax.iota(jnp.int32, 16) + cb) & (T - 1)
        gh[p] = pltpu.async_copy(x_hbm.at[idxs[p]], rows[p], sem_g)

    start_gather(0)
    for k in range(_NCH):
        p = k & 1
        gh[p].wait()
        sh[p] = pltpu.async_copy(rows[p], xs_hbm.at[dstb_v.at[k]], sem_s)
        if k + 1 < _NCH:
            if k + 1 >= 2:
                sh[(k + 1) & 1].wait()
            start_gather(k + 1)
    sh[(_NCH - 1) & 1].wait()
    sh[(_NCH - 2) & 1].wait()

    @pl.when(wid == 0)
    def _():
        pltpu.sync_copy(dst_hbm, dall_v)
        pltpu.sync_copy(wp_hbm, wall_v)

        def body(j, carry):
            idx = dall_v[pl.ds(j * 16, 16)]
            w = wall_v[pl.ds(j * 16, 16)]
            plsc.store_scatter(wbuf_v, [idx], w)
            return carry

        lax.fori_loop(0, (2 * T) // 16, body, 0)
        pltpu.sync_copy(wbuf_v, wrow_hbm)


def _dispatch(x, dst3, dst, wp):
    f = pl.kernel(
        _dispatch_body,
        out_type=[
            jax.ShapeDtypeStruct((MAXROWS, H), jnp.float32),
            jax.ShapeDtypeStruct((MAXROWS,), jnp.float32),
        ],
        mesh=plsc.VectorSubcoreMesh(core_axis_name="c", subcore_axis_name="s"),
        scratch_types=[
            pltpu.VMEM((_NCH, _CH), jnp.int32),
            pltpu.VMEM((_CH,), jnp.int32),
            pltpu.VMEM((_CH,), jnp.int32),
            pltpu.VMEM((_CH, H), jnp.float32),
            pltpu.VMEM((_CH, H), jnp.float32),
            pltpu.VMEM((2 * T,), jnp.int32),
            pltpu.VMEM((2 * T,), jnp.float32),
            pltpu.VMEM((MAXROWS,), jnp.float32),
            pltpu.SemaphoreType.DMA,
            pltpu.SemaphoreType.DMA,
        ],
        compiler_params=pltpu.CompilerParams(needs_layout_passes=False),
    )
    return f(x, dst3, dst, wp)


def _gateup_body(be_s, nb_s, xs_ref, wg_ref, wu_ref, a_ref):
    i = pl.program_id(0)

    @pl.when(i < nb_s[0])
    def _():
        x = xs_ref[...].astype(jnp.float32)
        g = jnp.dot(x, wg_ref[0], preferred_element_type=jnp.float32)
        u = jnp.dot(x, wu_ref[0], preferred_element_type=jnp.float32)
        a = (g / (1.0 + jnp.exp(-g))) * u
        a_ref[...] = a.astype(jnp.bfloat16)


def _down_body(be_s, nb_s, a_ref, wd_ref, wrow_ref, ys_ref):
    i = pl.program_id(0)

    @pl.when(i < nb_s[0])
    def _():
        a = a_ref[...].astype(jnp.float32)
        w = wrow_ref[0][0][:, None]
        y = jnp.dot(a, wd_ref[0], preferred_element_type=jnp.float32)
        ys_ref[...] = y * w


def _grouped(xs, wgu, wd, wrow, be, nb):
    vp = pltpu.CompilerParams(vmem_limit_bytes=63 * 1024 * 1024)
    act = pl.pallas_call(
        _gateup_body,
        grid_spec=pltpu.PrefetchScalarGridSpec(
            num_scalar_prefetch=2,
            grid=(MAXBLK,),
            in_specs=[
                pl.BlockSpec((BLK, H), lambda i, be, nb: (i, 0)),
                pl.BlockSpec((1, H, FFN), lambda i, be, nb: (be[i], 0, 0)),
                pl.BlockSpec((1, H, FFN), lambda i, be, nb: (be[i], 0, 1)),
            ],
            out_specs=pl.BlockSpec((BLK, FFN), lambda i, be, nb: (i, 0)),
        ),
        out_shape=jax.ShapeDtypeStruct((MAXROWS, FFN), jnp.bfloat16),
        compiler_params=vp,
    )(be, nb, xs, wgu, wgu)
    ys_halves = []
    for half in (0, 1):
        ys_halves.append(pl.pallas_call(
            _down_body,
            grid_spec=pltpu.PrefetchScalarGridSpec(
                num_scalar_prefetch=2,
                grid=(MAXBLK,),
                in_specs=[
                    pl.BlockSpec((BLK, FFN), lambda i, be, nb: (i, 0)),
                    pl.BlockSpec((1, FFN, H // 2),
                                 lambda i, be, nb, _h=half: (be[i], 0, _h)),
                    pl.BlockSpec((1, 1, BLK), lambda i, be, nb: (i, 0, 0)),
                ],
                out_specs=pl.BlockSpec((BLK, H // 2), lambda i, be, nb: (i, 0)),
            ),
            out_shape=jax.ShapeDtypeStruct((MAXROWS, H // 2), jnp.float32),
            compiler_params=vp,
        )(be, nb, act, wd, wrow))
    return ys_halves


_SBLK = 128


def _shared_body(x_ref, wg_ref, wu_ref, wd_ref, out0_ref, out1_ref):
    x = x_ref[...]
    g = jnp.dot(x, wg_ref[...], preferred_element_type=jnp.float32)
    u = jnp.dot(x, wu_ref[...], preferred_element_type=jnp.float32)
    a = (g / (1.0 + jnp.exp(-g))) * u
    y = jnp.dot(a, wd_ref[...], preferred_element_type=jnp.float32)
    out0_ref[...] = y[:, :H // 2]
    out1_ref[...] = y[:, H // 2:]


def _shared(x, wgu, wd):
    single = pl.Buffered(buffer_count=1)
    return pl.pallas_call(
        _shared_body,
        grid=(T // _SBLK,),
        in_specs=[
            pl.BlockSpec((_SBLK, H), lambda i: (i, 0)),
            pl.BlockSpec((H, FFN), lambda i: (0, 0), pipeline_mode=single),
            pl.BlockSpec((H, FFN), lambda i: (0, 1), pipeline_mode=single),
            pl.BlockSpec((FFN, H), lambda i: (0, 0), pipeline_mode=single),
        ],
        out_specs=[pl.BlockSpec((_SBLK, H // 2), lambda i: (i, 0)),
                   pl.BlockSpec((_SBLK, H // 2), lambda i: (i, 0))],
        out_shape=[jax.ShapeDtypeStruct((T, H // 2), jnp.float32),
                   jax.ShapeDtypeStruct((T, H // 2), jnp.float32)],
        compiler_params=pltpu.CompilerParams(
            vmem_limit_bytes=63 * 1024 * 1024),
    )(x, wgu, wgu, wd)


_CCH = 8
_CNCH = T // 32 // _CCH


_HH = H // 2


def _combine_body(ys_hbm, sh_hbm, dst_hbm, out_hbm,
                  r1_v, r2_v, y1a, y1b, y2a, y2b, sha, shb, sem_g, sem_s):
    cid = lax.axis_index("c")
    sid = lax.axis_index("s")
    wid = sid * 2 + cid
    ntok = T // 32
    base = wid * ntok
    pltpu.sync_copy(dst_hbm.at[pl.ds(base, ntok)], r1_v)
    pltpu.sync_copy(dst_hbm.at[pl.ds(T + base, ntok)], r2_v)
    y1 = (y1a, y1b)
    y2 = (y2a, y2b)
    shv = (sha, shb)
    gh, oh = {}, {}

    def start(k):
        p = k & 1
        tb = base + k * _CCH
        gh[p] = (
            pltpu.async_copy(ys_hbm.at[r1_v.at[pl.ds(k * _CCH, _CCH)]],
                             y1[p], sem_g),
            pltpu.async_copy(ys_hbm.at[r2_v.at[pl.ds(k * _CCH, _CCH)]],
                             y2[p], sem_g),
            pltpu.async_copy(sh_hbm.at[pl.ds(tb, _CCH)], shv[p], sem_g),
        )

    start(0)
    for k in range(_CNCH):
        p = k & 1
        for h in gh[p]:
            h.wait()
        if k + 1 < _CNCH:
            if k >= 1:
                oh[(k + 1) & 1].wait()
            start(k + 1)
        for i in range(_CCH):
            def body(j, carry, _i=i, _p=p):
                sl = pl.ds(j * 16, 16)
                shv[_p][_i, sl] = (shv[_p][_i, sl]
                                   + y1[_p][_i, sl] + y2[_p][_i, sl])
                return carry
            lax.fori_loop(0, _HH // 16, body, 0, unroll=4)
        oh[p] = pltpu.async_copy(shv[p], out_hbm.at[pl.ds(base + k * _CCH,
                                                          _CCH)], sem_s)
    oh[(_CNCH - 1) & 1].wait()
    oh[(_CNCH - 2) & 1].wait()


def _combine(ys, shared, dst):
    f = pl.kernel(
        _combine_body,
        out_type=jax.ShapeDtypeStruct((T, _HH), jnp.float32),
        mesh=plsc.VectorSubcoreMesh(core_axis_name="c", subcore_axis_name="s"),
        scratch_types=[
            pltpu.VMEM((T // 32,), jnp.int32),
            pltpu.VMEM((T // 32,), jnp.int32),
            pltpu.VMEM((_CCH, _HH), jnp.float32),
            pltpu.VMEM((_CCH, _HH), jnp.float32),
            pltpu.VMEM((_CCH, _HH), jnp.float32),
            pltpu.VMEM((_CCH, _HH), jnp.float32),
            pltpu.VMEM((_CCH, _HH), jnp.float32),
            pltpu.VMEM((_CCH, _HH), jnp.float32),
            pltpu.SemaphoreType.DMA,
            pltpu.SemaphoreType.DMA,
        ],
        compiler_params=pltpu.CompilerParams(needs_layout_passes=False),
    )
    return f(ys, shared, dst)


def kernel(hidden_states, gate_w, expert_gate_up, expert_down,
           shared_gate_up, shared_down):
    b, s, h = hidden_states.shape
    x = hidden_states.reshape(T, H)
    dst2d, wp2d, be2d, nb2d = _router(x, gate_w)
    dst = dst2d.reshape(2 * T)
    wp = wp2d.reshape(2 * T)
    be = be2d.reshape(MAXBLK)
    nb = nb2d.reshape(1)
    xs, wrow = _dispatch(x, dst.reshape(_NCH * 32, _CH), dst, wp)
    shared_lo, shared_hi = _shared(x, shared_gate_up, shared_down)
    ys_lo, ys_hi = _grouped(xs, expert_gate_up, expert_down,
                            wrow.reshape(MAXBLK, 1, BLK), be, nb)
    out_lo = _combine(ys_lo, shared_lo, dst)
    out_hi = _combine(ys_hi, shared_hi, dst)
    out = jnp.concatenate([out_lo, out_hi], axis=1)
    return out.reshape(b, s, h)


# speedup vs baseline: 1.0700x; 1.0700x over previous
import functools

import jax
import jax.numpy as jnp
from jax import lax
from jax.experimental import pallas as pl
from jax.experimental.pallas import tpu as pltpu
from jax.experimental.pallas import tpu_sc as plsc

T = 2048
H = 2048
FFN = 1408
E = 16
NG = 4
GSZ = E // NG
TOPK = 2
SCALE = 1.0

BLK = 128
MAXBLK = (T * TOPK) // BLK + E
MAXROWS = MAXBLK * BLK

NEG = -1e30


def _router_body(x_ref, gw_ref, dst_ref, wp_ref, be_ref, nb_ref):
    x = x_ref[...]
    gw = gw_ref[...]
    logits = lax.dot_general(x, gw, (((1,), (1,)), ((), ())),
                             preferred_element_type=jnp.float32)
    col = lax.broadcasted_iota(jnp.int32, (T, E), 1)
    gid = col // GSZ
    gmax_b = jnp.zeros((T, E), jnp.float32)
    for g in range(NG):
        m = jnp.max(jnp.where(gid == g, logits, NEG), axis=1, keepdims=True)
        gmax_b = jnp.where(gid == g, m, gmax_b)
    v1 = jnp.max(gmax_b, axis=1, keepdims=True)
    c1 = jnp.min(jnp.where(gmax_b == v1, col, E), axis=1, keepdims=True)
    g1 = c1 // GSZ
    masked2 = jnp.where(gid == g1, NEG, gmax_b)
    v2 = jnp.max(masked2, axis=1, keepdims=True)
    c2 = jnp.min(jnp.where(masked2 == v2, col, E), axis=1, keepdims=True)
    g2 = c2 // GSZ
    group_mask = (gid == g1) | (gid == g2)
    s = jnp.where(group_mask, logits, 0.0)
    m = jnp.max(s, axis=1, keepdims=True)
    ex = jnp.exp(s - m)
    p = ex / jnp.sum(ex, axis=1, keepdims=True)
    w1 = jnp.max(p, axis=1, keepdims=True)
    i1 = jnp.min(jnp.where(p == w1, col, E), axis=1, keepdims=True)
    p2 = jnp.where(col == i1, -1.0, p)
    w2 = jnp.max(p2, axis=1, keepdims=True)
    i2 = jnp.min(jnp.where(p2 == w2, col, E), axis=1, keepdims=True)

    oh1 = (i1 == col).astype(jnp.float32)
    oh2 = (i2 == col).astype(jnp.float32)
    oh = jnp.concatenate([oh1, oh2], axis=0)
    C = 128
    NC = (2 * T) // C
    r = lax.broadcasted_iota(jnp.int32, (C, C), 0)
    c = lax.broadcasted_iota(jnp.int32, (C, C), 1)
    tri = (c < r).astype(jnp.float32)
    locals_, totals_ = [], []
    for k in range(NC):
        chunk = oh[k * C:(k + 1) * C]
        locals_.append(jnp.dot(tri, chunk, preferred_element_type=jnp.float32))
        totals_.append(jnp.sum(chunk, axis=0, keepdims=True))
    totals = jnp.concatenate(totals_, axis=0)
    rr = lax.broadcasted_iota(jnp.int32, (NC, NC), 0)
    cc = lax.broadcasted_iota(jnp.int32, (NC, NC), 1)
    tri2 = (cc < rr).astype(jnp.float32)
    pref = jnp.dot(tri2, totals, preferred_element_type=jnp.float32)
    cum_ex = jnp.concatenate(
        [locals_[k] + pref[k:k + 1, :] for k in range(NC)], axis=0)
    rank = jnp.sum(cum_ex * oh, axis=1, keepdims=True)
    counts = jnp.sum(oh, axis=0, keepdims=True)
    padded = jnp.ceil(counts / BLK) * BLK
    jr = lax.broadcasted_iota(jnp.int32, (E, E), 0)
    ec = lax.broadcasted_iota(jnp.int32, (E, E), 1)
    off_r = jnp.sum(jnp.where(jr < ec, padded[0][:, None], 0.0),
                    axis=0, keepdims=True)
    ends_r = off_r + padded
    total_rows = jnp.sum(padded)
    dst = jnp.sum(off_r * oh, axis=1, keepdims=True) + rank
    dst_ref[...] = dst.astype(jnp.int32)
    wp_ref[...] = jnp.concatenate([w1, w2], axis=0) * SCALE
    nb = (total_rows / BLK).astype(jnp.int32)
    bi = lax.broadcasted_iota(jnp.int32, (MAXBLK, E), 0)
    be_raw = jnp.sum((ends_r <= bi.astype(jnp.float32) * BLK).astype(jnp.int32),
                     axis=1, keepdims=True)
    brow = lax.broadcasted_iota(jnp.int32, (MAXBLK, 1), 0)
    last_e = jnp.sum(jnp.where(brow == nb - 1, be_raw, 0))
    be_ref[...] = jnp.where(brow < nb, be_raw, last_e)
    nb_ref[...] = jnp.full((1, 1), nb, jnp.int32)


def _router(x, gate_w):
    return pl.pallas_call(
        _router_body,
        out_shape=[
            jax.ShapeDtypeStruct((2 * T, 1), jnp.int32),
            jax.ShapeDtypeStruct((2 * T, 1), jnp.float32),
            jax.ShapeDtypeStruct((MAXBLK, 1), jnp.int32),
            jax.ShapeDtypeStruct((1, 1), jnp.int32),
        ],
        compiler_params=pltpu.CompilerParams(
            vmem_limit_bytes=100 * 1024 * 1024),
    )(x, gate_w)


_CH = 16


_NCH = (2 * T) // 32 // _CH


def _dispatch_body(x_hbm, dst3_hbm, dst_hbm, wp_hbm, xs_hbm, wrow_hbm,
                   dstb_v, idx0_v, idx1_v, rows0_v, rows1_v,
                   dall_v, wall_v, wbuf_v, sem_g, sem_s):
    cid = lax.axis_index("c")
    sid = lax.axis_index("s")
    wid = sid * 2 + cid
    base = wid * _NCH * _CH
    pltpu.sync_copy(dst3_hbm.at[pl.ds(wid * _NCH, _NCH)], dstb_v)
    rows = (rows0_v, rows1_v)
    idxs = (idx0_v, idx1_v)
    gh, sh = {}, {}

    def start_gather(k):
        p = k & 1
        cb = base + k * _CH
        idxs[p][...] = (lax.iota(jnp.int32, 16) + cb) & (T - 1)
        gh[p] = pltpu.async_copy(x_hbm.at[idxs[p]], rows[p], sem_g)

    start_gather(0)
    for k in range(_NCH):
        p = k & 1
        gh[p].wait()
        sh[p] = pltpu.async_copy(rows[p], xs_hbm.at[dstb_v.at[k]], sem_s)
        if k + 1 < _NCH:
            if k + 1 >= 2:
                sh[(k + 1) & 1].wait()
            start_gather(k + 1)
    sh[(_NCH - 1) & 1].wait()
    sh[(_NCH - 2) & 1].wait()

    @pl.when(wid == 0)
    def _():
        pltpu.sync_copy(dst_hbm, dall_v)
        pltpu.sync_copy(wp_hbm, wall_v)

        def body(j, carry):
            idx = dall_v[pl.ds(j * 16, 16)]
            w = wall_v[pl.ds(j * 16, 16)]
            plsc.store_scatter(wbuf_v, [idx], w)
            return carry

        lax.fori_loop(0, (2 * T) // 16, body, 0)
        pltpu.sync_copy(wbuf_v, wrow_hbm)


def _dispatch(x, dst3, dst, wp):
    f = pl.kernel(
        _dispatch_body,
        out_type=[
            jax.ShapeDtypeStruct((MAXROWS, H), jnp.float32),
            jax.ShapeDtypeStruct((MAXROWS,), jnp.float32),
        ],
        mesh=plsc.VectorSubcoreMesh(core_axis_name="c", subcore_axis_name="s"),
        scratch_types=[
            pltpu.VMEM((_NCH, _CH), jnp.int32),
            pltpu.VMEM((_CH,), jnp.int32),
            pltpu.VMEM((_CH,), jnp.int32),
            pltpu.VMEM((_CH, H), jnp.float32),
            pltpu.VMEM((_CH, H), jnp.float32),
            pltpu.VMEM((2 * T,), jnp.int32),
            pltpu.VMEM((2 * T,), jnp.float32),
            pltpu.VMEM((MAXROWS,), jnp.float32),
            pltpu.SemaphoreType.DMA,
            pltpu.SemaphoreType.DMA,
        ],
        compiler_params=pltpu.CompilerParams(needs_layout_passes=False),
    )
    return f(x, dst3, dst, wp)


def _gateup_body(be_s, nb_s, xs_ref, wg_ref, wu_ref, a_ref):
    i = pl.program_id(0)

    @pl.when(i < nb_s[0])
    def _():
        x = xs_ref[...].astype(jnp.float32)
        g = jnp.dot(x, wg_ref[0], preferred_element_type=jnp.float32)
        u = jnp.dot(x, wu_ref[0], preferred_element_type=jnp.float32)
        a = (g / (1.0 + jnp.exp(-g))) * u
        a_ref[...] = a.astype(jnp.bfloat16)


def _down_body(be_s, nb_s, a_ref, wd0_ref, wd1_ref, wrow_ref, ys_ref):
    i = pl.program_id(0)

    @pl.when(i < nb_s[0])
    def _():
        a = a_ref[...].astype(jnp.float32)
        w = wrow_ref[0][0][:, None]
        y0 = jnp.dot(a, wd0_ref[0], preferred_element_type=jnp.float32)
        y1 = jnp.dot(a, wd1_ref[0], preferred_element_type=jnp.float32)
        ys_ref[...] = jnp.concatenate([y0, y1], axis=1) * w


def _grouped(xs, wgu, wd, wrow, be, nb):
    vp = pltpu.CompilerParams(vmem_limit_bytes=63 * 1024 * 1024)
    act = pl.pallas_call(
        _gateup_body,
        grid_spec=pltpu.PrefetchScalarGridSpec(
            num_scalar_prefetch=2,
            grid=(MAXBLK,),
            in_specs=[
                pl.BlockSpec((BLK, H), lambda i, be, nb: (i, 0)),
                pl.BlockSpec((1, H, FFN), lambda i, be, nb: (be[i], 0, 0)),
                pl.BlockSpec((1, H, FFN), lambda i, be, nb: (be[i], 0, 1)),
            ],
            out_specs=pl.BlockSpec((BLK, FFN), lambda i, be, nb: (i, 0)),
        ),
        out_shape=jax.ShapeDtypeStruct((MAXROWS, FFN), jnp.bfloat16),
        compiler_params=vp,
    )(be, nb, xs, wgu, wgu)
    return pl.pallas_call(
        _down_body,
        grid_spec=pltpu.PrefetchScalarGridSpec(
            num_scalar_prefetch=2,
            grid=(MAXBLK,),
            in_specs=[
                pl.BlockSpec((BLK, FFN), lambda i, be, nb: (i, 0)),
                pl.BlockSpec((1, FFN, H // 2), lambda i, be, nb: (be[i], 0, 0)),
                pl.BlockSpec((1, FFN, H // 2), lambda i, be, nb: (be[i], 0, 1)),
                pl.BlockSpec((1, 1, BLK), lambda i, be, nb: (i, 0, 0)),
            ],
            out_specs=pl.BlockSpec((BLK, H), lambda i, be, nb: (i, 0)),
        ),
        out_shape=jax.ShapeDtypeStruct((MAXROWS, H), jnp.float32),
        compiler_params=vp,
    )(be, nb, act, wd, wd, wrow)


_SBLK = 128


def _shared_body(x_ref, wg_ref, wu_ref, wd_ref, out_ref):
    x = x_ref[...]
    g = jnp.dot(x, wg_ref[...], preferred_element_type=jnp.float32)
    u = jnp.dot(x, wu_ref[...], preferred_element_type=jnp.float32)
    a = (g / (1.0 + jnp.exp(-g))) * u
    out_ref[...] = jnp.dot(a, wd_ref[...], preferred_element_type=jnp.float32)


def _shared(x, wgu, wd):
    single = pl.Buffered(buffer_count=1)
    return pl.pallas_call(
        _shared_body,
        grid=(T // _SBLK,),
        in_specs=[
            pl.BlockSpec((_SBLK, H), lambda i: (i, 0)),
            pl.BlockSpec((H, FFN), lambda i: (0, 0), pipeline_mode=single),
            pl.BlockSpec((H, FFN), lambda i: (0, 1), pipeline_mode=single),
            pl.BlockSpec((FFN, H), lambda i: (0, 0), pipeline_mode=single),
        ],
        out_specs=pl.BlockSpec((_SBLK, H), lambda i: (i, 0)),
        out_shape=jax.ShapeDtypeStruct((T, H), jnp.float32),
        compiler_params=pltpu.CompilerParams(
            vmem_limit_bytes=63 * 1024 * 1024),
    )(x, wgu, wgu, wd)


_CCH = 8
_CNCH = T // 32 // _CCH


def _combine_body(ys_hbm, sh_hbm, dst_hbm, out_hbm,
                  r1_v, r2_v, y1a, y1b, y2a, y2b, sha, shb, sem_g, sem_s):
    cid = lax.axis_index("c")
    sid = lax.axis_index("s")
    wid = sid * 2 + cid
    ntok = T // 32
    base = wid * ntok
    pltpu.sync_copy(dst_hbm.at[pl.ds(base, ntok)], r1_v)
    pltpu.sync_copy(dst_hbm.at[pl.ds(T + base, ntok)], r2_v)
    y1 = (y1a, y1b)
    y2 = (y2a, y2b)
    shv = (sha, shb)
    gh, oh = {}, {}

    def start(k):
        p = k & 1
        tb = base + k * _CCH
        gh[p] = (
            pltpu.async_copy(ys_hbm.at[r1_v.at[pl.ds(k * _CCH, _CCH)]],
                             y1[p], sem_g),
            pltpu.async_copy(ys_hbm.at[r2_v.at[pl.ds(k * _CCH, _CCH)]],
                             y2[p], sem_g),
            pltpu.async_copy(sh_hbm.at[pl.ds(tb, _CCH)], shv[p], sem_g),
        )

    start(0)
    for k in range(_CNCH):
        p = k & 1
        for h in gh[p]:
            h.wait()
        if k + 1 < _CNCH:
            if k >= 1:
                oh[(k + 1) & 1].wait()
            start(k + 1)
        for i in range(_CCH):
            def body(j, carry, _i=i, _p=p):
                sl = pl.ds(j * 16, 16)
                shv[_p][_i, sl] = (shv[_p][_i, sl]
                                   + y1[_p][_i, sl] + y2[_p][_i, sl])
                return carry
            lax.fori_loop(0, H // 16, body, 0, unroll=4)
        oh[p] = pltpu.async_copy(shv[p], out_hbm.at[pl.ds(base + k * _CCH,
                                                          _CCH)], sem_s)
    oh[(_CNCH - 1) & 1].wait()
    oh[(_CNCH - 2) & 1].wait()


def _combine(ys, shared, dst):
    f = pl.kernel(
        _combine_body,
        out_type=jax.ShapeDtypeStruct((T, H), jnp.float32),
        mesh=plsc.VectorSubcoreMesh(core_axis_name="c", subcore_axis_name="s"),
        scratch_types=[
            pltpu.VMEM((T // 32,), jnp.int32),
            pltpu.VMEM((T // 32,), jnp.int32),
            pltpu.VMEM((_CCH, H), jnp.float32),
            pltpu.VMEM((_CCH, H), jnp.float32),
            pltpu.VMEM((_CCH, H), jnp.float32),
            pltpu.VMEM((_CCH, H), jnp.float32),
            pltpu.VMEM((_CCH, H), jnp.float32),
            pltpu.VMEM((_CCH, H), jnp.float32),
            pltpu.SemaphoreType.DMA,
            pltpu.SemaphoreType.DMA,
        ],
        compiler_params=pltpu.CompilerParams(needs_layout_passes=False),
    )
    return f(ys, shared, dst)


def kernel(hidden_states, gate_w, expert_gate_up, expert_down,
           shared_gate_up, shared_down):
    b, s, h = hidden_states.shape
    x = hidden_states.reshape(T, H)
    dst2d, wp2d, be2d, nb2d = _router(x, gate_w)
    dst = dst2d.reshape(2 * T)
    wp = wp2d.reshape(2 * T)
    be = be2d.reshape(MAXBLK)
    nb = nb2d.reshape(1)
    xs, wrow = _dispatch(x, dst.reshape(_NCH * 32, _CH), dst, wp)
    shared = _shared(x, shared_gate_up, shared_down)
    ys = _grouped(xs, expert_gate_up, expert_down,
                  wrow.reshape(MAXBLK, 1, BLK), be, nb)
    out = _combine(ys, shared, dst)
    return out.reshape(b, s, h)
